# baseline (speedup 1.0000x reference)
import jax
import jax.numpy as jnp
from jax.experimental import pallas as pl
from jax.experimental.pallas import tpu as pltpu

DIM = 1024
MEM = 256
NEXP = 4
L = 512
KSZ = 5
DIL = 2
CHUNK = 32
NCH = L // CHUNK
EPS = 1e-6


def _dot_nt(a, b):
    return jax.lax.dot_general(a, b, (((1,), (1,)), ((), ())),
                               preferred_element_type=jnp.float32)


def _stage1_kernel(x_ref, norm_w_ref, dwT_ref, dw_b_ref, pw_ref, pw_b_ref,
                   val_w_ref, val_b_ref, td_ref, gate_w_ref, gate_b_ref,
                   h2_ref, nm_ref):
    x = x_ref[0]
    ms = jnp.mean(x * x, axis=-1, keepdims=True)
    h0 = x * jax.lax.rsqrt(ms + EPS) * norm_w_ref[...]
    pad = (KSZ - 1) * DIL
    hpad = jnp.concatenate([jnp.zeros((pad, DIM), jnp.float32), h0], axis=0)
    acc = h0 * dwT_ref[KSZ - 1:KSZ, :] + dw_b_ref[...]
    for t in range(KSZ - 1):
        off = t * DIL
        acc = acc + hpad[off:off + L, :] * dwT_ref[t:t + 1, :]
    h1 = jax.nn.silu(_dot_nt(acc, pw_ref[...]) + pw_b_ref[...])
    v = _dot_nt(h1, val_w_ref[...]) + val_b_ref[...]
    td = jax.nn.sigmoid(td_ref[...]) * 0.9 + 0.1
    ltd = jnp.log(td)
    ii = jax.lax.broadcasted_iota(jnp.int32, (CHUNK, CHUNK, 1), 0)
    jj = jax.lax.broadcasted_iota(jnp.int32, (CHUNK, CHUNK, 1), 1)
    diff = ii - jj
    mask = jnp.where(diff >= 0,
                     jnp.exp(diff.astype(jnp.float32)
                             * ltd[0][None, None, :]), 0.0)
    ivec = jax.lax.broadcasted_iota(jnp.int32, (CHUNK, 1), 0).astype(jnp.float32)
    powi = jnp.exp((ivec + 1.0) * ltd)
    rev = jnp.exp((CHUNK - 1.0 - ivec) * ltd)
    tdC = jnp.exp(CHUNK * ltd)
    carry = jnp.zeros((1, MEM), jnp.float32)
    parts = []
    for c in range(NCH):
        vch = v[c * CHUNK:(c + 1) * CHUNK, :]
        w_intra = jnp.sum(mask * vch[None, :, :], axis=1)
        parts.append(w_intra + powi * carry)
        carry = tdC * carry + jnp.sum(rev * vch, axis=0, keepdims=True)
    weighted = jnp.concatenate(parts, axis=0)
    gw = gate_w_ref[...]
    h2 = (h1 + _dot_nt(h1, gw[:, :DIM]) + _dot_nt(weighted, gw[:, DIM:])
          + gate_b_ref[...])
    h2_ref[0] = h2
    nm_ref[0] = weighted[L - 1:L, :]


def _stage2_kernel(x_ref, h2_ref, rout_w_ref, rout_b_ref, ew_ref, eb_ref,
                   op_w_ref, op_b_ref, fnorm_ref, down_w_ref, down_b_ref,
                   up_w_ref, up_b_ref, rs_ref, out_ref):
    h2 = h2_ref[0]
    logits = _dot_nt(h2, rout_w_ref[...]) + rout_b_ref[...]
    idx = jax.lax.broadcasted_iota(jnp.int32, (L, NEXP), 1)
    v1 = jnp.max(logits, axis=1, keepdims=True)
    i1 = jnp.min(jnp.where(logits >= v1, idx, NEXP), axis=1, keepdims=True)
    masked = jnp.where(idx == i1, -jnp.float32(3e38), logits)
    v2 = jnp.max(masked, axis=1, keepdims=True)
    i2 = jnp.min(jnp.where(masked >= v2, idx, NEXP), axis=1, keepdims=True)
    e2 = jnp.exp(v2 - v1)
    rw1 = 1.0 / (1.0 + e2)
    rw2 = e2 * rw1
    h2b = h2.astype(jnp.bfloat16)
    comb = jnp.zeros((L, DIM), jnp.float32)
    for e in range(NEXP):
        eo = jax.nn.silu(_dot_nt(h2b, ew_ref[e]) + eb_ref[e:e + 1, :])
        we = (rw1 * (i1 == e).astype(jnp.float32)
              + rw2 * (i2 == e).astype(jnp.float32))
        comb = comb + we * eo
    fo = _dot_nt(comb.astype(jnp.bfloat16), op_w_ref[...]) + op_b_ref[...]
    hb = h2 + fo
    ms = jnp.mean(hb * hb, axis=-1, keepdims=True)
    h3 = hb * jax.lax.rsqrt(ms + EPS) * fnorm_ref[...]
    z = (_dot_nt(h3.astype(jnp.bfloat16), down_w_ref[...])
         + down_b_ref[...])
    dn = 0.5 * z * (1.0 + jax.lax.erf(z * 0.7071067811865476))
    ff = _dot_nt(dn.astype(jnp.bfloat16), up_w_ref[...]) + up_b_ref[...]
    out_ref[0] = rs_ref[...] * x_ref[0] + h3 + ff


def _row(a):
    return a.reshape(1, -1)


def kernel(x, params):
    p = params
    B = x.shape[0]
    dwT = jnp.transpose(p['dw_w'][:, 0, :])

    def bs2(arr):
        return pl.BlockSpec(arr.shape, lambda b: (0, 0))

    def bs3(arr):
        return pl.BlockSpec(arr.shape, lambda b: (0, 0, 0))

    xspec = pl.BlockSpec((1, L, DIM), lambda b: (b, 0, 0))

    h2, nm = pl.pallas_call(
        _stage1_kernel,
        grid=(B,),
        in_specs=[
            xspec,
            bs2(_row(p['norm_w'])), bs2(dwT), bs2(_row(p['dw_b'])),
            bs2(p['pw_w'][:, :, 0]), bs2(_row(p['pw_b'])),
            bs2(p['val_w']), bs2(_row(p['val_b'])),
            bs2(_row(p['time_decay'])),
            bs2(p['gate_w']), bs2(_row(p['gate_b'])),
        ],
        out_specs=[
            pl.BlockSpec((1, L, DIM), lambda b: (b, 0, 0)),
            pl.BlockSpec((1, 1, MEM), lambda b: (b, 0, 0)),
        ],
        out_shape=[
            jax.ShapeDtypeStruct((B, L, DIM), jnp.float32),
            jax.ShapeDtypeStruct((B, 1, MEM), jnp.float32),
        ],
    )(x, _row(p['norm_w']), dwT, _row(p['dw_b']), p['pw_w'][:, :, 0],
      _row(p['pw_b']), p['val_w'], _row(p['val_b']), _row(p['time_decay']),
      p['gate_w'], _row(p['gate_b']))

    out = pl.pallas_call(
        _stage2_kernel,
        grid=(B,),
        in_specs=[
            xspec,
            pl.BlockSpec((1, L, DIM), lambda b: (b, 0, 0)),
            bs2(p['router_w']), bs2(_row(p['router_b'])),
            bs3(p['expert_w']), bs2(p['expert_b']),
            bs2(p['outproj_w']), bs2(_row(p['outproj_b'])),
            bs2(_row(p['fusion_norm_w'])),
            bs2(p['down_w']), bs2(_row(p['down_b'])),
            bs2(p['up_w']), bs2(_row(p['up_b'])),
            bs2(p['residual_scale'].reshape(1, 1)),
        ],
        out_specs=pl.BlockSpec((1, L, DIM), lambda b: (b, 0, 0)),
        out_shape=jax.ShapeDtypeStruct((B, L, DIM), jnp.float32),
    )(x, h2, p['router_w'], _row(p['router_b']),
      p['expert_w'].astype(jnp.bfloat16), p['expert_b'],
      p['outproj_w'].astype(jnp.bfloat16), _row(p['outproj_b']),
      _row(p['fusion_norm_w']),
      p['down_w'].astype(jnp.bfloat16), _row(p['down_b']),
      p['up_w'].astype(jnp.bfloat16), _row(p['up_b']),
      p['residual_scale'].reshape(1, 1))

    return out, nm.reshape(B, MEM)


# speedup vs baseline: 1.1881x; 1.1881x over previous
import jax
import jax.numpy as jnp
from jax.experimental import pallas as pl
from jax.experimental.pallas import tpu as pltpu

DIM = 1024
MEM = 256
NEXP = 4
B = 2
L = 512
N = B * L
KSZ = 5
DIL = 2
CHUNK = 32
NCH = L // CHUNK
EPS = 1e-6


def _dot_nt(a, b):
    return jax.lax.dot_general(a, b, (((1,), (1,)), ((), ())),
                               preferred_element_type=jnp.float32)


def _stage1_kernel(x_ref, norm_w_ref, dwT_ref, dw_b_ref, pw_ref, pw_b_ref,
                   val_w_ref, val_b_ref, td_ref, gate_w_ref, gate_b_ref,
                   h2_ref, nm_ref):
    pad = (KSZ - 1) * DIL
    accs = []
    for b in range(B):
        x = x_ref[b]
        ms = jnp.mean(x * x, axis=-1, keepdims=True)
        h0 = x * jax.lax.rsqrt(ms + EPS) * norm_w_ref[...]
        hpad = jnp.concatenate(
            [jnp.zeros((pad, DIM), jnp.float32), h0], axis=0)
        acc = h0 * dwT_ref[KSZ - 1:KSZ, :] + dw_b_ref[...]
        for t in range(KSZ - 1):
            off = t * DIL
            acc = acc + hpad[off:off + L, :] * dwT_ref[t:t + 1, :]
        accs.append(acc)
    acc = jnp.concatenate(accs, axis=0)
    h1 = jax.nn.silu(_dot_nt(acc, pw_ref[...]) + pw_b_ref[...])
    v = _dot_nt(h1, val_w_ref[...]) + val_b_ref[...]
    td = jax.nn.sigmoid(td_ref[...]) * 0.9 + 0.1
    ltd = jnp.log(td)
    ii = jax.lax.broadcasted_iota(jnp.int32, (CHUNK, CHUNK, 1), 0)
    jj = jax.lax.broadcasted_iota(jnp.int32, (CHUNK, CHUNK, 1), 1)
    diff = ii - jj
    mask = jnp.where(diff >= 0,
                     jnp.exp(diff.astype(jnp.float32)
                             * ltd[0][None, None, :]), 0.0)
    ivec = jax.lax.broadcasted_iota(jnp.int32, (CHUNK, 1), 0).astype(
        jnp.float32)
    powi = jnp.exp((ivec + 1.0) * ltd)
    rev = jnp.exp((CHUNK - 1.0 - ivec) * ltd)
    tdC = jnp.exp(CHUNK * ltd)
    parts = []
    for b in range(B):
        carry = jnp.zeros((1, MEM), jnp.float32)
        for c in range(NCH):
            vch = v[b * L + c * CHUNK:b * L + (c + 1) * CHUNK, :]
            w_intra = jnp.sum(mask * vch[None, :, :], axis=1)
            parts.append(w_intra + powi * carry)
            carry = tdC * carry + jnp.sum(rev * vch, axis=0, keepdims=True)
        nm_ref[b] = parts[-1][CHUNK - 1:CHUNK, :]
    weighted = jnp.concatenate(parts, axis=0)
    gw = gate_w_ref[...]
    h2 = (h1 + _dot_nt(h1, gw[:, :DIM]) + _dot_nt(weighted, gw[:, DIM:])
          + gate_b_ref[...])
    h2_ref[...] = h2.reshape(B, L, DIM)


def _stage2_kernel(x_ref, h2_ref, rout_w_ref, rout_b_ref, ew_ref, eb_ref,
                   op_w_ref, op_b_ref, fnorm_ref, down_w_ref, down_b_ref,
                   up_w_ref, up_b_ref, rs_ref, out_ref):
    h2 = h2_ref[...].reshape(N, DIM)
    logits = _dot_nt(h2, rout_w_ref[...]) + rout_b_ref[...]
    idx = jax.lax.broadcasted_iota(jnp.int32, (N, NEXP), 1)
    v1 = jnp.max(logits, axis=1, keepdims=True)
    i1 = jnp.min(jnp.where(logits >= v1, idx, NEXP), axis=1, keepdims=True)
    masked = jnp.where(idx == i1, -jnp.float32(3e38), logits)
    v2 = jnp.max(masked, axis=1, keepdims=True)
    i2 = jnp.min(jnp.where(masked >= v2, idx, NEXP), axis=1, keepdims=True)
    e2 = jnp.exp(v2 - v1)
    rw1 = 1.0 / (1.0 + e2)
    rw2 = e2 * rw1
    comb = jnp.zeros((N, DIM), jnp.float32)
    for e in range(NEXP):
        eo = jax.nn.silu(_dot_nt(h2, ew_ref[e]) + eb_ref[e:e + 1, :])
        we = (rw1 * (i1 == e).astype(jnp.float32)
              + rw2 * (i2 == e).astype(jnp.float32))
        comb = comb + we * eo
    fo = _dot_nt(comb, op_w_ref[...]) + op_b_ref[...]
    hb = h2 + fo
    ms = jnp.mean(hb * hb, axis=-1, keepdims=True)
    h3 = hb * jax.lax.rsqrt(ms + EPS) * fnorm_ref[...]
    z = _dot_nt(h3, down_w_ref[...]) + down_b_ref[...]
    dn = 0.5 * z * (1.0 + jax.lax.erf(z * 0.7071067811865476))
    ff = _dot_nt(dn, up_w_ref[...]) + up_b_ref[...]
    out = rs_ref[...] * x_ref[...].reshape(N, DIM) + h3 + ff
    out_ref[...] = out.reshape(B, L, DIM)


def _row(a):
    return a.reshape(1, -1)


def _full(arr):
    nd = arr.ndim
    return pl.BlockSpec(arr.shape, lambda: (0,) * nd)


def kernel(x, params):
    p = params
    dwT = jnp.transpose(p['dw_w'][:, 0, :])

    s1_in = [x, _row(p['norm_w']), dwT, _row(p['dw_b']), p['pw_w'][:, :, 0],
             _row(p['pw_b']), p['val_w'], _row(p['val_b']),
             _row(p['time_decay']), p['gate_w'], _row(p['gate_b'])]
    h2, nm = pl.pallas_call(
        _stage1_kernel,
        in_specs=[_full(a) for a in s1_in],
        out_specs=[
            pl.BlockSpec((B, L, DIM), lambda: (0, 0, 0)),
            pl.BlockSpec((B, 1, MEM), lambda: (0, 0, 0)),
        ],
        out_shape=[
            jax.ShapeDtypeStruct((B, L, DIM), jnp.float32),
            jax.ShapeDtypeStruct((B, 1, MEM), jnp.float32),
        ],
    )(*s1_in)

    s2_in = [x, h2, p['router_w'], _row(p['router_b']), p['expert_w'],
             p['expert_b'], p['outproj_w'], _row(p['outproj_b']),
             _row(p['fusion_norm_w']), p['down_w'], _row(p['down_b']),
             p['up_w'], _row(p['up_b']), p['residual_scale'].reshape(1, 1)]
    out = pl.pallas_call(
        _stage2_kernel,
        in_specs=[_full(a) for a in s2_in],
        out_specs=pl.BlockSpec((B, L, DIM), lambda: (0, 0, 0)),
        out_shape=jax.ShapeDtypeStruct((B, L, DIM), jnp.float32),
    )(*s2_in)

    return out, nm.reshape(B, MEM)


# speedup vs baseline: 1.3245x; 1.1148x over previous
import jax
import jax.numpy as jnp
from jax.experimental import pallas as pl
from jax.experimental.pallas import tpu as pltpu

DIM = 1024
MEM = 256
NEXP = 4
B = 2
L = 512
N = B * L
KSZ = 5
DIL = 2
CHUNK = 32
NCH = L // CHUNK
EPS = 1e-6


def _dot_nt(a, b):
    return jax.lax.dot_general(a, b, (((1,), (1,)), ((), ())),
                               preferred_element_type=jnp.float32)


def _kernel(x_ref, norm_w_ref, dwT_ref, dw_b_ref, pw_b_ref,
            val_w_ref, val_b_ref, td_ref, gate_b_ref,
            rout_w_ref, rout_b_ref, eb_ref, op_b_ref, fnorm_ref,
            down_w_ref, down_b_ref, up_w_ref, up_b_ref, rs_ref,
            pw_hbm, gate_hbm, ew_hbm, op_hbm,
            out_ref, nm_ref,
            pw_s, gate_s, ew_s, op_s,
            sem_pw, sem_gate, sem_ew, sem_op):
    cp_pw = pltpu.make_async_copy(pw_hbm, pw_s, sem_pw)
    cp_gate = pltpu.make_async_copy(gate_hbm, gate_s, sem_gate)
    cp_op = pltpu.make_async_copy(op_hbm, op_s, sem_op)

    def ew_copy(e):
        return pltpu.make_async_copy(ew_hbm.at[e], ew_s.at[e % 2],
                                     sem_ew.at[e % 2])

    cp_pw.start()
    cp_gate.start()
    ew_copy(0).start()
    ew_copy(1).start()
    cp_op.start()

    pad = (KSZ - 1) * DIL
    accs = []
    for b in range(B):
        x = x_ref[b]
        ms = jnp.mean(x * x, axis=-1, keepdims=True)
        h0 = x * jax.lax.rsqrt(ms + EPS) * norm_w_ref[...]
        hpad = jnp.concatenate(
            [jnp.zeros((pad, DIM), jnp.float32), h0], axis=0)
        acc = h0 * dwT_ref[KSZ - 1:KSZ, :] + dw_b_ref[...]
        for t in range(KSZ - 1):
            off = t * DIL
            acc = acc + hpad[off:off + L, :] * dwT_ref[t:t + 1, :]
        accs.append(acc)
    acc = jnp.concatenate(accs, axis=0)

    cp_pw.wait()
    h1 = jax.nn.silu(_dot_nt(acc, pw_s[...]) + pw_b_ref[...])
    v = _dot_nt(h1, val_w_ref[...]) + val_b_ref[...]

    td = jax.nn.sigmoid(td_ref[...]) * 0.9 + 0.1
    ltd = jnp.log(td)
    ii = jax.lax.broadcasted_iota(jnp.int32, (CHUNK, CHUNK, 1), 0)
    jj = jax.lax.broadcasted_iota(jnp.int32, (CHUNK, CHUNK, 1), 1)
    diff = ii - jj
    mask = jnp.where(diff >= 0,
                     jnp.exp(diff.astype(jnp.float32)
                             * ltd[0][None, None, :]), 0.0)
    ivec = jax.lax.broadcasted_iota(jnp.int32, (CHUNK, 1), 0).astype(
        jnp.float32)
    powi = jnp.exp((ivec + 1.0) * ltd)
    rev = jnp.exp((CHUNK - 1.0 - ivec) * ltd)
    tdC = jnp.exp(CHUNK * ltd)
    parts = []
    for b in range(B):
        carry = jnp.zeros((1, MEM), jnp.float32)
        for c in range(NCH):
            vch = v[b * L + c * CHUNK:b * L + (c + 1) * CHUNK, :]
            w_intra = jnp.sum(mask * vch[None, :, :], axis=1)
            parts.append(w_intra + powi * carry)
            carry = tdC * carry + jnp.sum(rev * vch, axis=0, keepdims=True)
        nm_ref[b] = parts[-1][CHUNK - 1:CHUNK, :]
    weighted = jnp.concatenate(parts, axis=0)

    cp_gate.wait()
    gw = gate_s[...]
    h2 = (h1 + _dot_nt(h1, gw[:, :DIM]) + _dot_nt(weighted, gw[:, DIM:])
          + gate_b_ref[...])

    logits = _dot_nt(h2, rout_w_ref[...]) + rout_b_ref[...]
    idx = jax.lax.broadcasted_iota(jnp.int32, (N, NEXP), 1)
    v1 = jnp.max(logits, axis=1, keepdims=True)
    i1 = jnp.min(jnp.where(logits >= v1, idx, NEXP), axis=1, keepdims=True)
    masked = jnp.where(idx == i1, -jnp.float32(3e38), logits)
    v2 = jnp.max(masked, axis=1, keepdims=True)
    i2 = jnp.min(jnp.where(masked >= v2, idx, NEXP), axis=1, keepdims=True)
    e2 = jnp.exp(v2 - v1)
    rw1 = 1.0 / (1.0 + e2)
    rw2 = e2 * rw1

    comb = jnp.zeros((N, DIM), jnp.float32)
    for e in range(NEXP):
        ew_copy(e).wait()
        eo = jax.nn.silu(_dot_nt(h2, ew_s[e % 2]) + eb_ref[e:e + 1, :])
        if e + 2 < NEXP:
            ew_copy(e + 2).start()
        we = (rw1 * (i1 == e).astype(jnp.float32)
              + rw2 * (i2 == e).astype(jnp.float32))
        comb = comb + we * eo

    cp_op.wait()
    fo = _dot_nt(comb, op_s[...]) + op_b_ref[...]
    hb = h2 + fo
    ms = jnp.mean(hb * hb, axis=-1, keepdims=True)
    h3 = hb * jax.lax.rsqrt(ms + EPS) * fnorm_ref[...]
    z = _dot_nt(h3, down_w_ref[...]) + down_b_ref[...]
    dn = 0.5 * z * (1.0 + jax.lax.erf(z * 0.7071067811865476))
    ff = _dot_nt(dn, up_w_ref[...]) + up_b_ref[...]
    out = rs_ref[...] * x_ref[...].reshape(N, DIM) + h3 + ff
    out_ref[...] = out.reshape(B, L, DIM)


def _row(a):
    return a.reshape(1, -1)


def _vspec(arr):
    nd = arr.ndim
    return pl.BlockSpec(arr.shape, lambda: (0,) * nd)


def kernel(x, params):
    p = params
    dwT = jnp.transpose(p['dw_w'][:, 0, :])

    vmem_in = [x, _row(p['norm_w']), dwT, _row(p['dw_b']), _row(p['pw_b']),
               p['val_w'], _row(p['val_b']), _row(p['time_decay']),
               _row(p['gate_b']), p['router_w'], _row(p['router_b']),
               p['expert_b'], _row(p['outproj_b']),
               _row(p['fusion_norm_w']), p['down_w'], _row(p['down_b']),
               p['up_w'], _row(p['up_b']), p['residual_scale'].reshape(1, 1)]
    hbm_in = [p['pw_w'][:, :, 0], p['gate_w'], p['expert_w'],
              p['outproj_w']]
    out, nm = pl.pallas_call(
        _kernel,
        in_specs=([_vspec(a) for a in vmem_in]
                  + [pl.BlockSpec(memory_space=pl.ANY)] * len(hbm_in)),
        out_specs=[
            pl.BlockSpec((B, L, DIM), lambda: (0, 0, 0)),
            pl.BlockSpec((B, 1, MEM), lambda: (0, 0, 0)),
        ],
        out_shape=[
            jax.ShapeDtypeStruct((B, L, DIM), jnp.float32),
            jax.ShapeDtypeStruct((B, 1, MEM), jnp.float32),
        ],
        scratch_shapes=[
            pltpu.VMEM((DIM, DIM), jnp.float32),
            pltpu.VMEM((DIM, DIM + MEM), jnp.float32),
            pltpu.VMEM((2, DIM, DIM), jnp.float32),
            pltpu.VMEM((DIM, DIM), jnp.float32),
            pltpu.SemaphoreType.DMA,
            pltpu.SemaphoreType.DMA,
            pltpu.SemaphoreType.DMA((2,)),
            pltpu.SemaphoreType.DMA,
        ],
    )(*vmem_in, *hbm_in)

    return out, nm.reshape(B, MEM)


# speedup vs baseline: 1.3468x; 1.0169x over previous
import jax
import jax.numpy as jnp
from jax.experimental import pallas as pl
from jax.experimental.pallas import tpu as pltpu

DIM = 1024
MEM = 256
NEXP = 4
B = 2
L = 512
N = B * L
KSZ = 5
DIL = 2
CHUNK = 16
NCH = L // CHUNK
EPS = 1e-6


def _dot_nt(a, b):
    return jax.lax.dot_general(a, b, (((1,), (1,)), ((), ())),
                               preferred_element_type=jnp.float32)


def _kernel(x_ref, norm_w_ref, dwT_ref, dw_b_ref, pw_b_ref,
            val_w_ref, val_b_ref, td_ref, gate_b_ref,
            rout_w_ref, rout_b_ref, eb_ref, op_b_ref, fnorm_ref,
            down_w_ref, down_b_ref, up_w_ref, up_b_ref, rs_ref,
            pw_hbm, gate_hbm, ew_hbm, op_hbm,
            out_ref, nm_ref,
            pw_s, gate_s, ew_s, op_s,
            sem_pw, sem_gate, sem_ew, sem_op):
    cp_pw = pltpu.make_async_copy(pw_hbm, pw_s, sem_pw)
    cp_gate = pltpu.make_async_copy(gate_hbm, gate_s, sem_gate)
    cp_op = pltpu.make_async_copy(op_hbm, op_s, sem_op)

    def ew_copy(e):
        return pltpu.make_async_copy(ew_hbm.at[e], ew_s.at[e % 2],
                                     sem_ew.at[e % 2])

    cp_pw.start()
    cp_gate.start()
    ew_copy(0).start()
    ew_copy(1).start()
    cp_op.start()

    pad = (KSZ - 1) * DIL
    accs = []
    for b in range(B):
        x = x_ref[b]
        ms = jnp.mean(x * x, axis=-1, keepdims=True)
        h0 = x * jax.lax.rsqrt(ms + EPS) * norm_w_ref[...]
        hpad = jnp.concatenate(
            [jnp.zeros((pad, DIM), jnp.float32), h0], axis=0)
        acc = h0 * dwT_ref[KSZ - 1:KSZ, :] + dw_b_ref[...]
        for t in range(KSZ - 1):
            off = t * DIL
            acc = acc + hpad[off:off + L, :] * dwT_ref[t:t + 1, :]
        accs.append(acc)
    acc = jnp.concatenate(accs, axis=0)

    cp_pw.wait()
    h1 = jax.nn.silu(_dot_nt(acc, pw_s[...]) + pw_b_ref[...])
    v = _dot_nt(h1, val_w_ref[...]) + val_b_ref[...]

    td = jax.nn.sigmoid(td_ref[...]) * 0.9 + 0.1
    ltd = jnp.log(td)
    ii = jax.lax.broadcasted_iota(jnp.int32, (CHUNK, CHUNK, 1), 0)
    jj = jax.lax.broadcasted_iota(jnp.int32, (CHUNK, CHUNK, 1), 1)
    diff = ii - jj
    mask = jnp.where(diff >= 0,
                     jnp.exp(diff.astype(jnp.float32)
                             * ltd[0][None, None, :]), 0.0)
    ivec = jax.lax.broadcasted_iota(jnp.int32, (CHUNK, 1), 0).astype(
        jnp.float32)
    powi = jnp.exp((ivec + 1.0) * ltd)
    rev = jnp.exp((CHUNK - 1.0 - ivec) * ltd)
    tdC = jnp.exp(CHUNK * ltd)
    parts = []
    for b in range(B):
        carry = jnp.zeros((1, MEM), jnp.float32)
        for c in range(NCH):
            vch = v[b * L + c * CHUNK:b * L + (c + 1) * CHUNK, :]
            w_intra = jnp.sum(mask * vch[None, :, :], axis=1)
            parts.append(w_intra + powi * carry)
            carry = tdC * carry + jnp.sum(rev * vch, axis=0, keepdims=True)
        nm_ref[b] = parts[-1][CHUNK - 1:CHUNK, :]
    weighted = jnp.concatenate(parts, axis=0)

    cp_gate.wait()
    gw = gate_s[...]
    h2 = (h1 + _dot_nt(h1, gw[:, :DIM]) + _dot_nt(weighted, gw[:, DIM:])
          + gate_b_ref[...])

    logits = _dot_nt(h2, rout_w_ref[...]) + rout_b_ref[...]
    idx = jax.lax.broadcasted_iota(jnp.int32, (N, NEXP), 1)
    v1 = jnp.max(logits, axis=1, keepdims=True)
    i1 = jnp.min(jnp.where(logits >= v1, idx, NEXP), axis=1, keepdims=True)
    masked = jnp.where(idx == i1, -jnp.float32(3e38), logits)
    v2 = jnp.max(masked, axis=1, keepdims=True)
    i2 = jnp.min(jnp.where(masked >= v2, idx, NEXP), axis=1, keepdims=True)
    e2 = jnp.exp(v2 - v1)
    rw1 = 1.0 / (1.0 + e2)
    rw2 = e2 * rw1

    h2b = h2.astype(jnp.bfloat16)
    comb = jnp.zeros((N, DIM), jnp.float32)
    for e in range(NEXP):
        ew_copy(e).wait()
        eo = jax.nn.silu(_dot_nt(h2b, ew_s[e % 2].astype(jnp.bfloat16))
                         + eb_ref[e:e + 1, :])
        if e + 2 < NEXP:
            ew_copy(e + 2).start()
        we = (rw1 * (i1 == e).astype(jnp.float32)
              + rw2 * (i2 == e).astype(jnp.float32))
        comb = comb + we * eo

    cp_op.wait()
    fo = (_dot_nt(comb.astype(jnp.bfloat16), op_s[...].astype(jnp.bfloat16))
          + op_b_ref[...])
    hb = h2 + fo
    ms = jnp.mean(hb * hb, axis=-1, keepdims=True)
    h3 = hb * jax.lax.rsqrt(ms + EPS) * fnorm_ref[...]
    z = (_dot_nt(h3.astype(jnp.bfloat16), down_w_ref[...].astype(jnp.bfloat16))
         + down_b_ref[...])
    dn = 0.5 * z * (1.0 + jax.lax.erf(z * 0.7071067811865476))
    ff = (_dot_nt(dn.astype(jnp.bfloat16), up_w_ref[...].astype(jnp.bfloat16))
          + up_b_ref[...])
    out = rs_ref[...] * x_ref[...].reshape(N, DIM) + h3 + ff
    out_ref[...] = out.reshape(B, L, DIM)


def _row(a):
    return a.reshape(1, -1)


def _vspec(arr):
    nd = arr.ndim
    return pl.BlockSpec(arr.shape, lambda: (0,) * nd)


def kernel(x, params):
    p = params
    dwT = jnp.transpose(p['dw_w'][:, 0, :])

    vmem_in = [x, _row(p['norm_w']), dwT, _row(p['dw_b']), _row(p['pw_b']),
               p['val_w'], _row(p['val_b']), _row(p['time_decay']),
               _row(p['gate_b']), p['router_w'], _row(p['router_b']),
               p['expert_b'], _row(p['outproj_b']),
               _row(p['fusion_norm_w']), p['down_w'], _row(p['down_b']),
               p['up_w'], _row(p['up_b']), p['residual_scale'].reshape(1, 1)]
    hbm_in = [p['pw_w'][:, :, 0], p['gate_w'], p['expert_w'],
              p['outproj_w']]
    out, nm = pl.pallas_call(
        _kernel,
        in_specs=([_vspec(a) for a in vmem_in]
                  + [pl.BlockSpec(memory_space=pl.ANY)] * len(hbm_in)),
        out_specs=[
            pl.BlockSpec((B, L, DIM), lambda: (0, 0, 0)),
            pl.BlockSpec((B, 1, MEM), lambda: (0, 0, 0)),
        ],
        out_shape=[
            jax.ShapeDtypeStruct((B, L, DIM), jnp.float32),
            jax.ShapeDtypeStruct((B, 1, MEM), jnp.float32),
        ],
        scratch_shapes=[
            pltpu.VMEM((DIM, DIM), jnp.float32),
            pltpu.VMEM((DIM, DIM + MEM), jnp.float32),
            pltpu.VMEM((2, DIM, DIM), jnp.float32),
            pltpu.VMEM((DIM, DIM), jnp.float32),
            pltpu.SemaphoreType.DMA,
            pltpu.SemaphoreType.DMA,
            pltpu.SemaphoreType.DMA((2,)),
            pltpu.SemaphoreType.DMA,
        ],
    )(*vmem_in, *hbm_in)

    return out, nm.reshape(B, MEM)


# speedup vs baseline: 1.4178x; 1.0527x over previous
import jax
import jax.numpy as jnp
from jax.experimental import pallas as pl
from jax.experimental.pallas import tpu as pltpu

DIM = 1024
MEM = 256
NEXP = 4
B = 2
L = 512
N = B * L
KSZ = 5
DIL = 2
CHUNK = 32
NCH = L // CHUNK
EPS = 1e-6


def _dot_nt(a, b):
    return jax.lax.dot_general(a, b, (((1,), (1,)), ((), ())),
                               preferred_element_type=jnp.float32)


def _kernel(x_ref, norm_w_ref, dwT_ref, dw_b_ref, pw_b_ref,
            val_b_ref, td_ref, gate_b_ref,
            rout_w_ref, rout_b_ref, eb_ref, op_b_ref, fnorm_ref,
            down_b_ref, up_b_ref, rs_ref,
            pw_hbm, gate_hbm, ew_hbm, op_hbm, val_hbm, down_hbm, up_hbm,
            out_ref, nm_ref,
            pw_s, gate_s, ew_s, op_s, val_s, down_s, up_s,
            sem_pw, sem_gate, sem_ew, sem_op, sem_sm):
    cp_pw = pltpu.make_async_copy(pw_hbm, pw_s, sem_pw)
    cp_gate = pltpu.make_async_copy(gate_hbm, gate_s, sem_gate)
    cp_op = pltpu.make_async_copy(op_hbm, op_s, sem_op)
    cp_val = pltpu.make_async_copy(val_hbm, val_s, sem_sm.at[0])
    cp_down = pltpu.make_async_copy(down_hbm, down_s, sem_sm.at[1])
    cp_up = pltpu.make_async_copy(up_hbm, up_s, sem_sm.at[2])

    def ew_copy(e):
        return pltpu.make_async_copy(ew_hbm.at[e], ew_s.at[e % 2],
                                     sem_ew.at[e % 2])

    cp_pw.start()
    cp_val.start()
    cp_gate.start()
    ew_copy(0).start()
    ew_copy(1).start()
    cp_op.start()
    cp_down.start()
    cp_up.start()

    pad = (KSZ - 1) * DIL
    accs = []
    for b in range(B):
        x = x_ref[b]
        ms = jnp.mean(x * x, axis=-1, keepdims=True)
        h0 = x * jax.lax.rsqrt(ms + EPS) * norm_w_ref[...]
        hpad = jnp.concatenate(
            [jnp.zeros((pad, DIM), jnp.float32), h0], axis=0)
        acc = h0 * dwT_ref[KSZ - 1:KSZ, :] + dw_b_ref[...]
        for t in range(KSZ - 1):
            off = t * DIL
            acc = acc + hpad[off:off + L, :] * dwT_ref[t:t + 1, :]
        accs.append(acc)
    acc = jnp.concatenate(accs, axis=0)

    cp_pw.wait()
    h1 = jax.nn.silu(_dot_nt(acc, pw_s[...]) + pw_b_ref[...])
    cp_val.wait()
    v = _dot_nt(h1, val_s[...]) + val_b_ref[...]

    td = jax.nn.sigmoid(td_ref[...]) * 0.9 + 0.1
    ltd = jnp.log(td)
    ivec = jax.lax.broadcasted_iota(jnp.int32, (CHUNK, 1), 0).astype(
        jnp.float32)
    ascale = jnp.exp(-ivec * ltd)
    pscale = jnp.exp(ivec * ltd)
    powi = pscale * td
    tdC = jnp.exp(CHUNK * ltd)
    tdC1 = jnp.exp((CHUNK - 1.0) * ltd)
    parts = []
    for b in range(B):
        carry = jnp.zeros((1, MEM), jnp.float32)
        for c in range(NCH):
            vch = v[b * L + c * CHUNK:b * L + (c + 1) * CHUNK, :]
            y = vch * ascale
            s = 1
            while s < CHUNK:
                y = y + jnp.concatenate(
                    [jnp.zeros((s, MEM), jnp.float32), y[:CHUNK - s]], axis=0)
                s *= 2
            parts.append(pscale * y + powi * carry)
            carry = tdC * carry + tdC1 * y[CHUNK - 1:CHUNK, :]
        nm_ref[b] = parts[-1][CHUNK - 1:CHUNK, :]
    weighted = jnp.concatenate(parts, axis=0)

    cp_gate.wait()
    gw = gate_s[...]
    h2 = (h1 + _dot_nt(h1, gw[:, :DIM]) + _dot_nt(weighted, gw[:, DIM:])
          + gate_b_ref[...])

    logits = _dot_nt(h2, rout_w_ref[...]) + rout_b_ref[...]
    idx = jax.lax.broadcasted_iota(jnp.int32, (N, NEXP), 1)
    v1 = jnp.max(logits, axis=1, keepdims=True)
    i1 = jnp.min(jnp.where(logits >= v1, idx, NEXP), axis=1, keepdims=True)
    masked = jnp.where(idx == i1, -jnp.float32(3e38), logits)
    v2 = jnp.max(masked, axis=1, keepdims=True)
    i2 = jnp.min(jnp.where(masked >= v2, idx, NEXP), axis=1, keepdims=True)
    e2 = jnp.exp(v2 - v1)
    rw1 = 1.0 / (1.0 + e2)
    rw2 = e2 * rw1

    h2b = h2.astype(jnp.bfloat16)
    comb = jnp.zeros((N, DIM), jnp.float32)
    for e in range(NEXP):
        ew_copy(e).wait()
        eo = jax.nn.silu(_dot_nt(h2b, ew_s[e % 2].astype(jnp.bfloat16))
                         + eb_ref[e:e + 1, :])
        if e + 2 < NEXP:
            ew_copy(e + 2).start()
        we = (rw1 * (i1 == e).astype(jnp.float32)
              + rw2 * (i2 == e).astype(jnp.float32))
        comb = comb + we * eo

    cp_op.wait()
    fo = (_dot_nt(comb.astype(jnp.bfloat16), op_s[...].astype(jnp.bfloat16))
          + op_b_ref[...])
    hb = h2 + fo
    ms = jnp.mean(hb * hb, axis=-1, keepdims=True)
    h3 = hb * jax.lax.rsqrt(ms + EPS) * fnorm_ref[...]
    cp_down.wait()
    z = (_dot_nt(h3.astype(jnp.bfloat16), down_s[...].astype(jnp.bfloat16))
         + down_b_ref[...])
    dn = 0.5 * z * (1.0 + jax.lax.erf(z * 0.7071067811865476))
    cp_up.wait()
    ff = (_dot_nt(dn.astype(jnp.bfloat16), up_s[...].astype(jnp.bfloat16))
          + up_b_ref[...])
    out = rs_ref[...] * x_ref[...].reshape(N, DIM) + h3 + ff
    out_ref[...] = out.reshape(B, L, DIM)


def _row(a):
    return a.reshape(1, -1)


def _vspec(arr):
    nd = arr.ndim
    return pl.BlockSpec(arr.shape, lambda: (0,) * nd)


def kernel(x, params):
    p = params
    dwT = jnp.transpose(p['dw_w'][:, 0, :])

    vmem_in = [x, _row(p['norm_w']), dwT, _row(p['dw_b']), _row(p['pw_b']),
               _row(p['val_b']), _row(p['time_decay']),
               _row(p['gate_b']), p['router_w'], _row(p['router_b']),
               p['expert_b'], _row(p['outproj_b']),
               _row(p['fusion_norm_w']), _row(p['down_b']),
               _row(p['up_b']), p['residual_scale'].reshape(1, 1)]
    hbm_in = [p['pw_w'][:, :, 0], p['gate_w'], p['expert_w'],
              p['outproj_w'], p['val_w'], p['down_w'], p['up_w']]
    out, nm = pl.pallas_call(
        _kernel,
        in_specs=([_vspec(a) for a in vmem_in]
                  + [pl.BlockSpec(memory_space=pl.ANY)] * len(hbm_in)),
        out_specs=[
            pl.BlockSpec((B, L, DIM), lambda: (0, 0, 0)),
            pl.BlockSpec((B, 1, MEM), lambda: (0, 0, 0)),
        ],
        out_shape=[
            jax.ShapeDtypeStruct((B, L, DIM), jnp.float32),
            jax.ShapeDtypeStruct((B, 1, MEM), jnp.float32),
        ],
        scratch_shapes=[
            pltpu.VMEM((DIM, DIM), jnp.float32),
            pltpu.VMEM((DIM, DIM + MEM), jnp.float32),
            pltpu.VMEM((2, DIM, DIM), jnp.float32),
            pltpu.VMEM((DIM, DIM), jnp.float32),
            pltpu.VMEM((MEM, DIM), jnp.float32),
            pltpu.VMEM((MEM, DIM), jnp.float32),
            pltpu.VMEM((DIM, MEM), jnp.float32),
            pltpu.SemaphoreType.DMA,
            pltpu.SemaphoreType.DMA,
            pltpu.SemaphoreType.DMA((2,)),
            pltpu.SemaphoreType.DMA,
            pltpu.SemaphoreType.DMA((3,)),
        ],
    )(*vmem_in, *hbm_in)

    return out, nm.reshape(B, MEM)
